# baseline (speedup 1.0000x reference)
import functools

import jax
import jax.numpy as jnp
from jax import lax
from jax.experimental import pallas as pl
from jax.experimental.pallas import tpu as pltpu
from jax.experimental.pallas import tpu_sc as plsc

N = 10000
E = 160000
D_IN = 256
D_H = 512

NP = 10240
EP = 172032
CHUNK = 112
NSL = 4
SLW = D_H // NSL
ROWS_PER_TILE = NP // 16
ZCH = 80
OCH = 128

_f32 = jnp.float32
_i32 = jnp.int32

_MESH = plsc.VectorSubcoreMesh(core_axis_name="c", subcore_axis_name="s")
_SC_PARAMS = pltpu.CompilerParams(needs_layout_passes=False)


def _sc_deg_body(dstp, ewp, deg2, dst_v, ew_v, zero_v, acc):
    c = lax.axis_index("c")
    ss = lax.axis_index("s")

    def _z(r, _):
        zero_v[pl.ds(r * 16, 16)] = jnp.zeros((16,), _f32)
        return 0
    lax.fori_loop(0, ROWS_PER_TILE // 16, _z, 0)
    pltpu.sync_copy(zero_v, acc.at[pl.ds(ss * ROWS_PER_TILE, ROWS_PER_TILE)])
    plsc.subcore_barrier()

    per_tile = EP // 32
    tid = c * 16 + ss

    def _chunk(g, _):
        base = tid * per_tile + g * CHUNK
        pltpu.sync_copy(dstp.at[pl.ds(base, CHUNK)], dst_v)
        pltpu.sync_copy(ewp.at[pl.ds(base, CHUNK)], ew_v)
        pltpu.sync_copy(ew_v, acc.at[dst_v], add=True)
        return 0
    lax.fori_loop(0, per_tile // CHUNK, _chunk, 0)
    plsc.subcore_barrier()

    pltpu.sync_copy(acc.at[pl.ds(ss * ROWS_PER_TILE, ROWS_PER_TILE)],
                    deg2.at[c, pl.ds(ss * ROWS_PER_TILE, ROWS_PER_TILE)])


_sc_deg = pl.kernel(
    _sc_deg_body,
    out_type=jax.ShapeDtypeStruct((2, NP), _f32),
    mesh=_MESH,
    compiler_params=_SC_PARAMS,
    scratch_types=[
        pltpu.VMEM((CHUNK,), _i32),
        pltpu.VMEM((CHUNK,), _f32),
        pltpu.VMEM((ROWS_PER_TILE,), _f32),
        pltpu.VMEM_SHARED((NP,), _f32),
    ],
)


N_CHUNKS = EP // 16 // CHUNK
N_TRIPLES = N_CHUNKS // 3
GROUP = 12


def _sc_agg_body(xw, srcp, dstp, ewp, agg,
                 src_g, dst_g, ew_g,
                 gA, gB, gC, dA, dB, dC, nA, nB, nC, rA, rB, rC,
                 sgA, sgB, sgC, ssA, ssB, ssC, acc):
    c = lax.axis_index("c")
    tid = lax.axis_index("s")
    per_tile = EP // 16
    base = tid * per_tile

    def _load_group(j):
        off = base + j * GROUP * CHUNK
        n = GROUP * CHUNK
        pltpu.sync_copy(srcp.at[pl.ds(off, n)], src_g)
        pltpu.sync_copy(dstp.at[pl.ds(off, n)], dst_g)
        pltpu.sync_copy(ewp.at[pl.ds(off, n)], ew_g)

    def _prep(gbuf, dbuf, nbuf, g, s):
        off = (g % GROUP) * CHUNK
        for k in range(CHUNK // 16):
            sl = pl.ds(k * 16, 16)
            go = pl.ds(off + k * 16, 16)
            gbuf[sl] = src_g[go] * NSL + s
            dbuf[sl] = dst_g[go]
            nbuf[sl] = ew_g[go]

    def _wait(sem, rows):
        pltpu.make_async_copy(xw.at[pl.ds(0, CHUNK)], rows, sem).wait()

    def _scale(rows, nbuf):
        def _row(r, _):
            sc16 = plsc.load_gather(nbuf, [jnp.full((16,), r, _i32)])
            for k in range(SLW // 16):
                sl = pl.ds(k * 16, 16)
                rows[r, sl] = rows[r, sl] * sc16
            return 0
        lax.fori_loop(0, CHUNK, _row, 0, unroll=2)

    for p in range(NSL // 2):
        s = c * (NSL // 2) + p

        def _z(r, _):
            for k in range(SLW // 16):
                rA[r, pl.ds(k * 16, 16)] = jnp.zeros((16,), _f32)
            return 0
        lax.fori_loop(0, ZCH, _z, 0)
        for k in range(ROWS_PER_TILE // ZCH):
            pltpu.sync_copy(
                rA.at[pl.ds(0, ZCH)],
                acc.at[pl.ds(tid * ROWS_PER_TILE + k * ZCH, ZCH)])
        plsc.subcore_barrier()

        _load_group(0)
        _prep(gA, dA, nA, 0, s)
        pltpu.async_copy(xw.at[gA], rA, sgA)
        _prep(gB, dB, nB, 1, s)
        pltpu.async_copy(xw.at[gB], rB, sgB)

        def _triple(i, _):
            g0 = 3 * i

            _wait(sgA, rA)
            _scale(rA, nA)
            pltpu.async_copy(rA, acc.at[dA], ssA, add=True)

            @pl.when((g0 + 2) % GROUP == 0)
            def _():
                _load_group((g0 + 2) // GROUP)
            _prep(gC, dC, nC, g0 + 2, s)

            @pl.when(i > 0)
            def _():
                _wait(ssC, rC)
            pltpu.async_copy(xw.at[gC], rC, sgC)

            _wait(sgB, rB)
            _scale(rB, nB)
            pltpu.async_copy(rB, acc.at[dB], ssB, add=True)

            @pl.when(i < N_TRIPLES - 1)
            def _():
                @pl.when((g0 + 3) % GROUP == 0)
                def _():
                    _load_group((g0 + 3) // GROUP)
                _prep(gA, dA, nA, g0 + 3, s)
                _wait(ssA, rA)
                pltpu.async_copy(xw.at[gA], rA, sgA)

            _wait(sgC, rC)
            _scale(rC, nC)
            pltpu.async_copy(rC, acc.at[dC], ssC, add=True)

            @pl.when(i < N_TRIPLES - 1)
            def _():
                @pl.when((g0 + 4) % GROUP == 0)
                def _():
                    _load_group((g0 + 4) // GROUP)
                _prep(gB, dB, nB, g0 + 4, s)
                _wait(ssB, rB)
                pltpu.async_copy(xw.at[gB], rB, sgB)
            return 0
        lax.fori_loop(0, N_TRIPLES, _triple, 0)

        _wait(ssA, rA)
        _wait(ssB, rB)
        _wait(ssC, rC)
        plsc.subcore_barrier()

        for k in range(ROWS_PER_TILE // OCH):
            r0 = tid * ROWS_PER_TILE + k * OCH
            pltpu.sync_copy(acc.at[pl.ds(r0, OCH)],
                            agg.at[s, pl.ds(r0, OCH)])
        plsc.subcore_barrier()


_sc_agg = pl.kernel(
    _sc_agg_body,
    out_type=jax.ShapeDtypeStruct((NSL, NP, SLW), _f32),
    mesh=_MESH,
    compiler_params=_SC_PARAMS,
    scratch_types=[
        pltpu.VMEM((GROUP * CHUNK,), _i32),
        pltpu.VMEM((GROUP * CHUNK,), _i32),
        pltpu.VMEM((GROUP * CHUNK,), _f32),
        pltpu.VMEM((CHUNK,), _i32),
        pltpu.VMEM((CHUNK,), _i32),
        pltpu.VMEM((CHUNK,), _i32),
        pltpu.VMEM((CHUNK,), _i32),
        pltpu.VMEM((CHUNK,), _i32),
        pltpu.VMEM((CHUNK,), _i32),
        pltpu.VMEM((CHUNK,), _f32),
        pltpu.VMEM((CHUNK,), _f32),
        pltpu.VMEM((CHUNK,), _f32),
        pltpu.VMEM((CHUNK, SLW), _f32),
        pltpu.VMEM((CHUNK, SLW), _f32),
        pltpu.VMEM((CHUNK, SLW), _f32),
        pltpu.SemaphoreType.DMA,
        pltpu.SemaphoreType.DMA,
        pltpu.SemaphoreType.DMA,
        pltpu.SemaphoreType.DMA,
        pltpu.SemaphoreType.DMA,
        pltpu.SemaphoreType.DMA,
        pltpu.VMEM_SHARED((NP, SLW), _f32),
    ],
)


def _dinv_body(deg_ref, dinv_ref):
    d = deg_ref[0] + deg_ref[1]
    safe = jnp.where(d > 0.0, d, 1.0)
    dinv_ref[...] = jnp.where(d > 0.0, lax.rsqrt(safe), 0.0)


def _tc_dinv(deg2):
    return pl.pallas_call(
        _dinv_body,
        out_shape=jax.ShapeDtypeStruct((NP // 128, 128), _f32),
    )(deg2.reshape(2, NP // 128, 128))


def _mm_raw_body(x_ref, w_ref, o_ref):
    o_ref[...] = jnp.dot(x_ref[...], w_ref[...],
                         preferred_element_type=_f32)


def _tc_mm_raw(xp, W):
    bn = 1024
    k = xp.shape[1]
    return pl.pallas_call(
        _mm_raw_body,
        grid=(NP // bn,),
        in_specs=[
            pl.BlockSpec((bn, k), lambda i: (i, 0)),
            pl.BlockSpec((k, D_H), lambda i: (0, 0)),
        ],
        out_specs=pl.BlockSpec((bn, D_H), lambda i: (i, 0)),
        out_shape=jax.ShapeDtypeStruct((NP, D_H), _f32),
    )(xp, W)


def _scale_body(y_ref, dv_ref, o_ref):
    o_ref[...] = dv_ref[...] * y_ref[...]


def _tc_scale(y, dinv):
    bn = 1024
    return pl.pallas_call(
        _scale_body,
        grid=(NP // bn,),
        in_specs=[
            pl.BlockSpec((bn, D_H), lambda i: (i, 0)),
            pl.BlockSpec((bn, 1), lambda i: (i, 0)),
        ],
        out_specs=pl.BlockSpec((bn, D_H), lambda i: (i, 0)),
        out_shape=jax.ShapeDtypeStruct((NP, D_H), _f32),
    )(y, dinv.reshape(NP, 1))


def _l2_body(agg_ref, b_ref, w_ref, dv_ref, o_ref):
    a = jnp.concatenate([agg_ref[s] for s in range(NSL)], axis=-1)
    h = jnp.maximum(dv_ref[...] * a + b_ref[...], 0.0)
    o_ref[...] = dv_ref[...] * jnp.dot(h, w_ref[...],
                                       preferred_element_type=_f32)


def _tc_l2(agg1, b1, W2, dinv):
    bn = 1024
    return pl.pallas_call(
        _l2_body,
        grid=(NP // bn,),
        in_specs=[
            pl.BlockSpec((NSL, bn, SLW), lambda i: (0, i, 0)),
            pl.BlockSpec((1, D_H), lambda i: (0, 0)),
            pl.BlockSpec((D_H, D_H), lambda i: (0, 0)),
            pl.BlockSpec((bn, 1), lambda i: (i, 0)),
        ],
        out_specs=pl.BlockSpec((bn, D_H), lambda i: (i, 0)),
        out_shape=jax.ShapeDtypeStruct((NP, D_H), _f32),
    )(agg1, b1.reshape(1, D_H), W2, dinv.reshape(NP, 1))


def _out_body(agg_ref, b_ref, dv_ref, o_ref):
    a = jnp.concatenate([agg_ref[s] for s in range(NSL)], axis=-1)
    o_ref[...] = dv_ref[...] * a + b_ref[...]


def _tc_out(agg2, b2, dinv):
    bn = 1024
    return pl.pallas_call(
        _out_body,
        grid=(NP // bn,),
        in_specs=[
            pl.BlockSpec((NSL, bn, SLW), lambda i: (0, i, 0)),
            pl.BlockSpec((1, D_H), lambda i: (0, 0)),
            pl.BlockSpec((bn, 1), lambda i: (i, 0)),
        ],
        out_specs=pl.BlockSpec((bn, D_H), lambda i: (i, 0)),
        out_shape=jax.ShapeDtypeStruct((NP, D_H), _f32),
    )(agg2, b2.reshape(1, D_H), dinv.reshape(NP, 1))


@jax.jit
def kernel(x, edge_index, edge_weight, W1, b1, W2, b2):
    src = edge_index[0]
    dst = edge_index[1]
    loop = jnp.arange(N, dtype=_i32)
    pad = jnp.arange(EP - E - N, dtype=_i32) % N
    srcp = jnp.concatenate([src, loop, pad])
    dstp = jnp.concatenate([dst, loop, pad])
    ewp = jnp.concatenate([
        edge_weight,
        jnp.ones((N,), _f32),
        jnp.zeros((EP - E - N,), _f32),
    ])
    xp = jnp.pad(x, ((0, NP - N), (0, 0)))

    deg2 = _sc_deg(dstp, ewp)
    y1r = _tc_mm_raw(xp, W1)
    dinv = _tc_dinv(deg2).reshape(NP)
    y1 = _tc_scale(y1r, dinv)
    agg1 = _sc_agg(y1.reshape(NP * NSL, SLW), srcp, dstp, ewp)
    y2 = _tc_l2(agg1, b1, W2, dinv)
    agg2 = _sc_agg(y2.reshape(NP * NSL, SLW), srcp, dstp, ewp)
    out = _tc_out(agg2, b2, dinv)
    return out[:N]


# speedup vs baseline: 1.0470x; 1.0470x over previous
import functools

import jax
import jax.numpy as jnp
from jax import lax
from jax.experimental import pallas as pl
from jax.experimental.pallas import tpu as pltpu
from jax.experimental.pallas import tpu_sc as plsc

N = 10000
E = 160000
D_IN = 256
D_H = 512

NP = 10240
EP = 172032
CHUNK = 112
NSL = 4
SLW = D_H // NSL
ROWS_PER_TILE = NP // 16
ZCH = 80
OCH = 128

_f32 = jnp.float32
_i32 = jnp.int32

_MESH = plsc.VectorSubcoreMesh(core_axis_name="c", subcore_axis_name="s")
_SC_PARAMS = pltpu.CompilerParams(needs_layout_passes=False)


def _sc_deg_body(dstp, ewp, deg2, dA, eA, dB, eB, zero_v, acc,
                 sdA, seA, sdB, seB):
    c = lax.axis_index("c")
    ss = lax.axis_index("s")

    per_tile = EP // 32
    nch = per_tile // CHUNK
    tid = c * 16 + ss

    def _load(g, dv, ev, sd, se):
        base = tid * per_tile + g * CHUNK
        pltpu.async_copy(dstp.at[pl.ds(base, CHUNK)], dv, sd)
        pltpu.async_copy(ewp.at[pl.ds(base, CHUNK)], ev, se)

    def _wait(dv, ev, sd, se):
        pltpu.make_async_copy(dstp.at[pl.ds(0, CHUNK)], dv, sd).wait()
        pltpu.make_async_copy(ewp.at[pl.ds(0, CHUNK)], ev, se).wait()

    _load(0, dA, eA, sdA, seA)
    _load(1, dB, eB, sdB, seB)

    def _z(r, _):
        zero_v[pl.ds(r * 16, 16)] = jnp.zeros((16,), _f32)
        return 0
    lax.fori_loop(0, ROWS_PER_TILE // 16, _z, 0)
    pltpu.sync_copy(zero_v, acc.at[pl.ds(ss * ROWS_PER_TILE, ROWS_PER_TILE)])
    plsc.subcore_barrier()

    def _pair(i, _):
        g0 = 2 * i
        _wait(dA, eA, sdA, seA)
        pltpu.sync_copy(eA, acc.at[dA], add=True)
        @pl.when(g0 + 2 < nch)
        def _():
            _load(g0 + 2, dA, eA, sdA, seA)
        _wait(dB, eB, sdB, seB)
        pltpu.sync_copy(eB, acc.at[dB], add=True)
        @pl.when(g0 + 3 < nch)
        def _():
            _load(g0 + 3, dB, eB, sdB, seB)
        return 0
    lax.fori_loop(0, nch // 2, _pair, 0)
    plsc.subcore_barrier()

    pltpu.sync_copy(acc.at[pl.ds(ss * ROWS_PER_TILE, ROWS_PER_TILE)],
                    deg2.at[c, pl.ds(ss * ROWS_PER_TILE, ROWS_PER_TILE)])


_sc_deg = pl.kernel(
    _sc_deg_body,
    out_type=jax.ShapeDtypeStruct((2, NP), _f32),
    mesh=_MESH,
    compiler_params=_SC_PARAMS,
    scratch_types=[
        pltpu.VMEM((CHUNK,), _i32),
        pltpu.VMEM((CHUNK,), _f32),
        pltpu.VMEM((CHUNK,), _i32),
        pltpu.VMEM((CHUNK,), _f32),
        pltpu.VMEM((ROWS_PER_TILE,), _f32),
        pltpu.VMEM_SHARED((NP,), _f32),
        pltpu.SemaphoreType.DMA,
        pltpu.SemaphoreType.DMA,
        pltpu.SemaphoreType.DMA,
        pltpu.SemaphoreType.DMA,
    ],
)


N_CHUNKS = EP // 16 // CHUNK
N_TRIPLES = N_CHUNKS // 3
GROUP = 12


def _sc_agg_body(xw, srcp, dstp, ewp, agg,
                 src_g, dst_g, ew_g,
                 gA, gB, gC, dA, dB, dC, nA, nB, nC, rA, rB, rC,
                 sgA, sgB, sgC, ssA, ssB, ssC, acc):
    c = lax.axis_index("c")
    tid = lax.axis_index("s")
    per_tile = EP // 16
    base = tid * per_tile

    def _load_group(j):
        off = base + j * GROUP * CHUNK
        n = GROUP * CHUNK
        pltpu.sync_copy(srcp.at[pl.ds(off, n)], src_g)
        pltpu.sync_copy(dstp.at[pl.ds(off, n)], dst_g)
        pltpu.sync_copy(ewp.at[pl.ds(off, n)], ew_g)

    def _prep(gbuf, dbuf, nbuf, g, s):
        off = (g % GROUP) * CHUNK
        for k in range(CHUNK // 16):
            sl = pl.ds(k * 16, 16)
            go = pl.ds(off + k * 16, 16)
            gbuf[sl] = src_g[go] * NSL + s
            dbuf[sl] = dst_g[go]
            nbuf[sl] = ew_g[go]

    def _wait(sem, rows):
        pltpu.make_async_copy(xw.at[pl.ds(0, CHUNK)], rows, sem).wait()

    def _scale(rows, nbuf):
        def _row(r, _):
            sc16 = plsc.load_gather(nbuf, [jnp.full((16,), r, _i32)])
            for k in range(SLW // 16):
                sl = pl.ds(k * 16, 16)
                rows[r, sl] = rows[r, sl] * sc16
            return 0
        lax.fori_loop(0, CHUNK, _row, 0, unroll=2)

    for p in range(NSL // 2):
        s = c * (NSL // 2) + p

        def _z(r, _):
            for k in range(SLW // 16):
                rA[r, pl.ds(k * 16, 16)] = jnp.zeros((16,), _f32)
            return 0
        lax.fori_loop(0, ZCH, _z, 0)
        for k in range(ROWS_PER_TILE // ZCH):
            pltpu.sync_copy(
                rA.at[pl.ds(0, ZCH)],
                acc.at[pl.ds(tid * ROWS_PER_TILE + k * ZCH, ZCH)])
        plsc.subcore_barrier()

        _load_group(0)
        _prep(gA, dA, nA, 0, s)
        pltpu.async_copy(xw.at[gA], rA, sgA)
        _prep(gB, dB, nB, 1, s)
        pltpu.async_copy(xw.at[gB], rB, sgB)

        def _triple(i, _):
            g0 = 3 * i

            _wait(sgA, rA)
            _scale(rA, nA)
            pltpu.async_copy(rA, acc.at[dA], ssA, add=True)

            @pl.when((g0 + 2) % GROUP == 0)
            def _():
                _load_group((g0 + 2) // GROUP)
            _prep(gC, dC, nC, g0 + 2, s)

            @pl.when(i > 0)
            def _():
                _wait(ssC, rC)
            pltpu.async_copy(xw.at[gC], rC, sgC)

            _wait(sgB, rB)
            _scale(rB, nB)
            pltpu.async_copy(rB, acc.at[dB], ssB, add=True)

            @pl.when(i < N_TRIPLES - 1)
            def _():
                @pl.when((g0 + 3) % GROUP == 0)
                def _():
                    _load_group((g0 + 3) // GROUP)
                _prep(gA, dA, nA, g0 + 3, s)
                _wait(ssA, rA)
                pltpu.async_copy(xw.at[gA], rA, sgA)

            _wait(sgC, rC)
            _scale(rC, nC)
            pltpu.async_copy(rC, acc.at[dC], ssC, add=True)

            @pl.when(i < N_TRIPLES - 1)
            def _():
                @pl.when((g0 + 4) % GROUP == 0)
                def _():
                    _load_group((g0 + 4) // GROUP)
                _prep(gB, dB, nB, g0 + 4, s)
                _wait(ssB, rB)
                pltpu.async_copy(xw.at[gB], rB, sgB)
            return 0
        lax.fori_loop(0, N_TRIPLES, _triple, 0)

        _wait(ssA, rA)
        _wait(ssB, rB)
        _wait(ssC, rC)
        plsc.subcore_barrier()

        for k in range(ROWS_PER_TILE // OCH):
            r0 = tid * ROWS_PER_TILE + k * OCH
            pltpu.sync_copy(acc.at[pl.ds(r0, OCH)],
                            agg.at[s, pl.ds(r0, OCH)])
        plsc.subcore_barrier()


_sc_agg = pl.kernel(
    _sc_agg_body,
    out_type=jax.ShapeDtypeStruct((NSL, NP, SLW), _f32),
    mesh=_MESH,
    compiler_params=_SC_PARAMS,
    scratch_types=[
        pltpu.VMEM((GROUP * CHUNK,), _i32),
        pltpu.VMEM((GROUP * CHUNK,), _i32),
        pltpu.VMEM((GROUP * CHUNK,), _f32),
        pltpu.VMEM((CHUNK,), _i32),
        pltpu.VMEM((CHUNK,), _i32),
        pltpu.VMEM((CHUNK,), _i32),
        pltpu.VMEM((CHUNK,), _i32),
        pltpu.VMEM((CHUNK,), _i32),
        pltpu.VMEM((CHUNK,), _i32),
        pltpu.VMEM((CHUNK,), _f32),
        pltpu.VMEM((CHUNK,), _f32),
        pltpu.VMEM((CHUNK,), _f32),
        pltpu.VMEM((CHUNK, SLW), _f32),
        pltpu.VMEM((CHUNK, SLW), _f32),
        pltpu.VMEM((CHUNK, SLW), _f32),
        pltpu.SemaphoreType.DMA,
        pltpu.SemaphoreType.DMA,
        pltpu.SemaphoreType.DMA,
        pltpu.SemaphoreType.DMA,
        pltpu.SemaphoreType.DMA,
        pltpu.SemaphoreType.DMA,
        pltpu.VMEM_SHARED((NP, SLW), _f32),
    ],
)


def _dinv_body(deg_ref, dinv_ref):
    d = deg_ref[0] + deg_ref[1]
    safe = jnp.where(d > 0.0, d, 1.0)
    dinv_ref[...] = jnp.where(d > 0.0, lax.rsqrt(safe), 0.0)


def _tc_dinv(deg2):
    return pl.pallas_call(
        _dinv_body,
        out_shape=jax.ShapeDtypeStruct((NP // 128, 128), _f32),
    )(deg2.reshape(2, NP // 128, 128))


def _mm_body(x_ref, w_ref, dv_ref, o_ref):
    o_ref[...] = dv_ref[...] * jnp.dot(x_ref[...], w_ref[...],
                                       preferred_element_type=_f32)


def _tc_mm(xp, W, dinv):
    bn = 1024
    k = xp.shape[1]
    return pl.pallas_call(
        _mm_body,
        grid=(NP // bn,),
        in_specs=[
            pl.BlockSpec((bn, k), lambda i: (i, 0)),
            pl.BlockSpec((k, D_H), lambda i: (0, 0)),
            pl.BlockSpec((bn, 1), lambda i: (i, 0)),
        ],
        out_specs=pl.BlockSpec((bn, D_H), lambda i: (i, 0)),
        out_shape=jax.ShapeDtypeStruct((NP, D_H), _f32),
    )(xp, W, dinv.reshape(NP, 1))


def _l2_body(agg_ref, b_ref, w_ref, dv_ref, o_ref):
    a = jnp.concatenate([agg_ref[s] for s in range(NSL)], axis=-1)
    h = jnp.maximum(dv_ref[...] * a + b_ref[...], 0.0)
    o_ref[...] = dv_ref[...] * jnp.dot(h, w_ref[...],
                                       preferred_element_type=_f32)


def _tc_l2(agg1, b1, W2, dinv):
    bn = 1024
    return pl.pallas_call(
        _l2_body,
        grid=(NP // bn,),
        in_specs=[
            pl.BlockSpec((NSL, bn, SLW), lambda i: (0, i, 0)),
            pl.BlockSpec((1, D_H), lambda i: (0, 0)),
            pl.BlockSpec((D_H, D_H), lambda i: (0, 0)),
            pl.BlockSpec((bn, 1), lambda i: (i, 0)),
        ],
        out_specs=pl.BlockSpec((bn, D_H), lambda i: (i, 0)),
        out_shape=jax.ShapeDtypeStruct((NP, D_H), _f32),
    )(agg1, b1.reshape(1, D_H), W2, dinv.reshape(NP, 1))


def _out_body(agg_ref, b_ref, dv_ref, o_ref):
    a = jnp.concatenate([agg_ref[s] for s in range(NSL)], axis=-1)
    o_ref[...] = dv_ref[...] * a + b_ref[...]


def _tc_out(agg2, b2, dinv):
    bn = 1024
    return pl.pallas_call(
        _out_body,
        grid=(NP // bn,),
        in_specs=[
            pl.BlockSpec((NSL, bn, SLW), lambda i: (0, i, 0)),
            pl.BlockSpec((1, D_H), lambda i: (0, 0)),
            pl.BlockSpec((bn, 1), lambda i: (i, 0)),
        ],
        out_specs=pl.BlockSpec((bn, D_H), lambda i: (i, 0)),
        out_shape=jax.ShapeDtypeStruct((NP, D_H), _f32),
    )(agg2, b2.reshape(1, D_H), dinv.reshape(NP, 1))


@jax.jit
def kernel(x, edge_index, edge_weight, W1, b1, W2, b2):
    src = edge_index[0]
    dst = edge_index[1]
    loop = jnp.arange(N, dtype=_i32)
    pad = jnp.arange(EP - E - N, dtype=_i32) % N
    srcp = jnp.concatenate([src, loop, pad])
    dstp = jnp.concatenate([dst, loop, pad])
    ewp = jnp.concatenate([
        edge_weight,
        jnp.ones((N,), _f32),
        jnp.zeros((EP - E - N,), _f32),
    ])
    xp = jnp.pad(x, ((0, NP - N), (0, 0)))

    deg2 = _sc_deg(dstp, ewp)
    dinv = _tc_dinv(deg2).reshape(NP)

    y1 = _tc_mm(xp, W1, dinv)
    agg1 = _sc_agg(y1.reshape(NP * NSL, SLW), srcp, dstp, ewp)
    y2 = _tc_l2(agg1, b1, W2, dinv)
    agg2 = _sc_agg(y2.reshape(NP * NSL, SLW), srcp, dstp, ewp)
    out = _tc_out(agg2, b2, dinv)
    return out[:N]
